# Initial kernel scaffold; baseline (speedup 1.0000x reference)
import jax
import jax.numpy as jnp
from jax.experimental import pallas as pl


def kernel(latent_vectors, Wq, bq, Wk, bk, Wv, bv, Wo, bo, ln1_g, ln1_b, Wg0, bg0, lng0, lnb0, Wg1, bg1, lng1, lnb1, W1_mrna, b1_mrna, W2_mrna, b2_mrna, W1_methylation, b1_methylation, W2_methylation, b2_methylation, W1_mirna, b1_mirna, W2_mirna, b2_mirna):
    raise NotImplementedError("write your pallas kernel here")


# speedup vs baseline: 19.9307x; 19.9307x over previous
import functools
import math

import jax
import jax.numpy as jnp
from jax.experimental import pallas as pl

N = 2048
D = 256
H = 4
HD = 64
KN = 5
RB = 256
NEG = -1e30


def _qkv_body(lv_ref, w_ref, b_ref, q_ref, k_ref, v_ref):
    qkv = jnp.dot(lv_ref[...], w_ref[...], preferred_element_type=jnp.float32)
    qkv = qkv + b_ref[...]
    q_ref[...] = qkv[:, 0:D]
    k_ref[...] = qkv[:, D:2 * D]
    v_ref[...] = qkv[:, 2 * D:3 * D]


def _attn_topk_body(q_ref, k_ref, topk_ref):
    q = q_ref[...]
    k = k_ref[...]
    scale = 1.0 / math.sqrt(HD)
    acc = jnp.zeros((RB, N), jnp.float32)
    for h in range(H):
        qh = q[:, h * HD:(h + 1) * HD]
        kh = k[:, h * HD:(h + 1) * HD]
        s = jax.lax.dot_general(qh, kh, (((1,), (1,)), ((), ())),
                                preferred_element_type=jnp.float32) * scale
        s = s - jnp.max(s, axis=1, keepdims=True)
        e = jnp.exp(s)
        acc = acc + e / jnp.sum(e, axis=1, keepdims=True)
    am = acc * (1.0 / H)
    col = jax.lax.broadcasted_iota(jnp.int32, (RB, N), 1).astype(jnp.float32)
    picks = []
    for _ in range(KN):
        vmax = jnp.max(am, axis=1, keepdims=True)
        imax = jnp.max(jnp.where(am >= vmax, col, -1.0), axis=1, keepdims=True)
        picks.append(imax)
        am = jnp.where(col == imax, NEG, am)
    topk_ref[...] = jnp.concatenate(picks[::-1], axis=1)


def _ln(x, g, b, eps=1e-3):
    m = jnp.mean(x, axis=-1, keepdims=True)
    v = jnp.mean((x - m) * (x - m), axis=-1, keepdims=True)
    return (x - m) / jnp.sqrt(v + eps) * g + b


def _tail_body(topk_ref, lv_ref, q_ref, k_ref, v_ref,
               wo_ref, bo_ref, ln1g_ref, ln1b_ref,
               wg0_ref, bg0_ref, lng0_ref, lnb0_ref,
               wg1_ref, bg1_ref, lng1_ref, lnb1_ref,
               w1m_ref, b1m_ref, w2m_ref, b2m_ref,
               w1y_ref, b1y_ref, w2y_ref, b2y_ref,
               w1r_ref, b1r_ref, w2r_ref, b2r_ref,
               o1_ref, o2_ref, o3_ref):
    topk = topk_ref[...]
    NE = N * KN
    iota_e = jax.lax.broadcasted_iota(jnp.int32, (1, NE), 1).astype(jnp.float32)

    def gatherc(idx_col):
        return jnp.sum(jnp.where(idx_col == iota_e, topk, 0.0),
                       axis=1, keepdims=True)

    def degc(val_col):
        return jnp.sum(jnp.where(topk == val_col, 1.0, 0.0),
                       axis=1, keepdims=True)

    r15 = jax.lax.broadcasted_iota(jnp.int32, (15, 1), 0)
    e2 = ((r15 // KN) + (r15 % KN) * N).astype(jnp.float32)
    s2 = gatherc(e2)
    norm2 = jax.lax.rsqrt(5.0 * degc(s2))

    c3 = jax.lax.broadcasted_iota(jnp.int32, (3, 1), 0).astype(jnp.float32)
    S1 = jnp.concatenate([c3, s2], axis=0)
    r90 = jax.lax.broadcasted_iota(jnp.int32, (90, 1), 0)
    rep18 = jnp.where((r90 // KN) == jax.lax.broadcasted_iota(jnp.int32, (1, 18), 1),
                      1.0, 0.0)
    S1rep = jnp.dot(rep18, S1, preferred_element_type=jnp.float32,
                    precision=jax.lax.Precision.HIGHEST)
    e1 = S1rep + ((r90 % KN) * N).astype(jnp.float32)
    s1 = gatherc(e1)
    norm1 = jax.lax.rsqrt(5.0 * degc(s1))

    S0 = jnp.concatenate([S1, s1], axis=0)
    iota_n = jax.lax.broadcasted_iota(jnp.int32, (1, N), 1).astype(jnp.float32)
    onehot0 = jnp.where(S0 == iota_n, 1.0, 0.0)
    lv_sel = jnp.dot(onehot0, lv_ref[...], preferred_element_type=jnp.float32,
                     precision=jax.lax.Precision.HIGHEST)
    q_sel = jnp.dot(onehot0, q_ref[...], preferred_element_type=jnp.float32,
                    precision=jax.lax.Precision.HIGHEST)

    k_all = k_ref[...]
    v_all = v_ref[...]
    scale = 1.0 / math.sqrt(HD)
    ctxs = []
    for h in range(H):
        qh = q_sel[:, h * HD:(h + 1) * HD]
        kh = k_all[:, h * HD:(h + 1) * HD]
        s = jax.lax.dot_general(qh, kh, (((1,), (1,)), ((), ())),
                                preferred_element_type=jnp.float32) * scale
        s = s - jnp.max(s, axis=1, keepdims=True)
        e = jnp.exp(s)
        p = e / jnp.sum(e, axis=1, keepdims=True)
        ctxs.append(jnp.dot(p, v_all[:, h * HD:(h + 1) * HD],
                            preferred_element_type=jnp.float32))
    ctx = jnp.concatenate(ctxs, axis=1)
    mha = jnp.dot(ctx, wo_ref[...], preferred_element_type=jnp.float32) + bo_ref[...]
    x0 = _ln(lv_sel + mha, ln1g_ref[...], ln1b_ref[...])

    red18 = jnp.where(jax.lax.broadcasted_iota(jnp.int32, (18, 90), 0) ==
                      (jax.lax.broadcasted_iota(jnp.int32, (18, 90), 1) // KN),
                      1.0, 0.0)
    agg1 = jnp.dot(red18, x0[18:108, :] * norm1, preferred_element_type=jnp.float32,
                   precision=jax.lax.Precision.HIGHEST)
    x1 = _ln(x0[0:18, :] + jnp.dot(agg1, wg0_ref[...], preferred_element_type=jnp.float32) + bg0_ref[...],
             lng0_ref[...], lnb0_ref[...])

    red3 = jnp.where(jax.lax.broadcasted_iota(jnp.int32, (3, 15), 0) ==
                     (jax.lax.broadcasted_iota(jnp.int32, (3, 15), 1) // KN),
                     1.0, 0.0)
    agg2 = jnp.dot(red3, x1[3:18, :] * norm2, preferred_element_type=jnp.float32,
                   precision=jax.lax.Precision.HIGHEST)
    x2 = _ln(x1[0:3, :] + jnp.dot(agg2, wg1_ref[...], preferred_element_type=jnp.float32) + bg1_ref[...],
             lng1_ref[...], lnb1_ref[...])

    for row, (w1, b1, w2, b2, out) in enumerate((
            (w1m_ref, b1m_ref, w2m_ref, b2m_ref, o1_ref),
            (w1y_ref, b1y_ref, w2y_ref, b2y_ref, o2_ref),
            (w1r_ref, b1r_ref, w2r_ref, b2r_ref, o3_ref))):
        hdn = jnp.maximum(
            jnp.dot(x2[row:row + 1, :], w1[...], preferred_element_type=jnp.float32) + b1[...],
            0.0)
        out[...] = jnp.dot(hdn, w2[...], preferred_element_type=jnp.float32) + b2[...]


@jax.jit
def kernel(latent_vectors, Wq, bq, Wk, bk, Wv, bv, Wo, bo, ln1_g, ln1_b,
           Wg0, bg0, lng0, lnb0, Wg1, bg1, lng1, lnb1,
           W1_mrna, b1_mrna, W2_mrna, b2_mrna,
           W1_methylation, b1_methylation, W2_methylation, b2_methylation,
           W1_mirna, b1_mirna, W2_mirna, b2_mirna):
    lv = latent_vectors
    wqkv = jnp.concatenate([Wq.reshape(D, D), Wk.reshape(D, D), Wv.reshape(D, D)], axis=1)
    bqkv = jnp.concatenate([bq.reshape(1, D), bk.reshape(1, D), bv.reshape(1, D)], axis=1)

    q_all, k_all, v_all = pl.pallas_call(
        _qkv_body,
        out_shape=[jax.ShapeDtypeStruct((N, D), jnp.float32)] * 3,
    )(lv, wqkv, bqkv)

    topk = pl.pallas_call(
        _attn_topk_body,
        grid=(N // RB,),
        in_specs=[
            pl.BlockSpec((RB, D), lambda i: (i, 0)),
            pl.BlockSpec((N, D), lambda i: (0, 0)),
        ],
        out_specs=pl.BlockSpec((RB, KN), lambda i: (i, 0)),
        out_shape=jax.ShapeDtypeStruct((N, KN), jnp.float32),
    )(q_all, k_all)

    topk_flat = topk.reshape(1, N * KN)

    o1, o2, o3 = pl.pallas_call(
        _tail_body,
        out_shape=[jax.ShapeDtypeStruct((1, 1000), jnp.float32),
                   jax.ShapeDtypeStruct((1, 2000), jnp.float32),
                   jax.ShapeDtypeStruct((1, 500), jnp.float32)],
    )(topk_flat, lv, q_all, k_all, v_all,
      Wo.reshape(D, D), bo.reshape(1, D), ln1_g.reshape(1, D), ln1_b.reshape(1, D),
      Wg0, bg0.reshape(1, D), lng0.reshape(1, D), lnb0.reshape(1, D),
      Wg1, bg1.reshape(1, D), lng1.reshape(1, D), lnb1.reshape(1, D),
      W1_mrna, b1_mrna.reshape(1, -1), W2_mrna, b2_mrna.reshape(1, -1),
      W1_methylation, b1_methylation.reshape(1, -1), W2_methylation, b2_methylation.reshape(1, -1),
      W1_mirna, b1_mirna.reshape(1, -1), W2_mirna, b2_mirna.reshape(1, -1))

    return (o1.reshape(-1), o2.reshape(-1), o3.reshape(-1))
